# baseline (speedup 1.0000x reference)
import functools

import jax
import jax.numpy as jnp
from jax import lax
from jax.experimental import pallas as pl
from jax.experimental.pallas import tpu as pltpu
from jax.experimental.pallas import tpu_sc as plsc

S = 8192
D = 1024
B = 4096
SPAD = 8448
NROWS = SPAD + B

NC = 2
NS = 16
NW = NC * NS
RPW = NROWS // NW
SCH = (40, 40, 40, 40, 40, 40, 40, 40, 40, 32)

BM = 128
WCH = 1056


def _sc_gather_body(table, ids, out, idx_v, rows0, rows1, rows2,
                    g0, g1, g2, w0, w1, w2):
    wid = lax.axis_index("s") * NC + lax.axis_index("c")
    base = wid * RPW
    pltpu.sync_copy(ids.at[pl.ds(base, RPW)], idx_v)
    bufs = (rows0, rows1, rows2)
    gsems = (g0, g1, g2)
    wsems = (w0, w1, w2)
    n = len(SCH)
    offs = [sum(SCH[:c]) for c in range(n)]
    g_cps = [None] * n
    wb_cps = [None] * n
    for c in range(n):
        if c >= 3:
            wb_cps[c - 3].wait()
        b = bufs[c % 3].at[pl.ds(0, SCH[c])]
        g_cps[c] = pltpu.make_async_copy(
            table.at[idx_v.at[pl.ds(offs[c], SCH[c])]], b, gsems[c % 3])
        g_cps[c].start()
        if c >= 1:
            g_cps[c - 1].wait()
            pb = bufs[(c - 1) % 3].at[pl.ds(0, SCH[c - 1])]
            wb_cps[c - 1] = pltpu.make_async_copy(
                pb, out.at[pl.ds(base + offs[c - 1], SCH[c - 1])],
                wsems[(c - 1) % 3])
            wb_cps[c - 1].start()
    g_cps[n - 1].wait()
    wb_cps[n - 1] = pltpu.make_async_copy(
        bufs[(n - 1) % 3].at[pl.ds(0, SCH[n - 1])],
        out.at[pl.ds(base + offs[n - 1], SCH[n - 1])], wsems[(n - 1) % 3])
    wb_cps[n - 1].start()
    for c in (n - 3, n - 2, n - 1):
        wb_cps[c].wait()


@functools.cache
def _sc_gather():
    return pl.kernel(
        _sc_gather_body,
        out_type=jax.ShapeDtypeStruct((NROWS, D), jnp.float32),
        mesh=plsc.VectorSubcoreMesh(core_axis_name="c", subcore_axis_name="s"),
        scratch_types=[
            pltpu.VMEM((RPW,), jnp.int32),
            pltpu.VMEM((40, D), jnp.float32),
            pltpu.VMEM((40, D), jnp.float32),
            pltpu.VMEM((40, D), jnp.float32),
            pltpu.SemaphoreType.DMA,
            pltpu.SemaphoreType.DMA,
            pltpu.SemaphoreType.DMA,
            pltpu.SemaphoreType.DMA,
            pltpu.SemaphoreType.DMA,
            pltpu.SemaphoreType.DMA,
        ],
    )


def _main_body(xbf_ref, whbm_ref, tw_ref, lab_ref, ids_ref, frq_ref, tf_ref,
               out_ref, wv_ref, stage_ref, stage2_ref, sem, sem2):
    i = pl.program_id(0)

    @pl.when(i == 0)
    def _():
        stages = (stage_ref, stage2_ref)
        sems = (sem, sem2)
        ncast = SPAD // WCH
        cps = [None] * ncast
        for c in range(ncast):
            cps[c] = pltpu.make_async_copy(
                whbm_ref.at[pl.ds(c * WCH, WCH)], stages[c % 2], sems[c % 2])
            cps[c].start()
            if c >= 1:
                cps[c - 1].wait()
                wv_ref[pl.ds((c - 1) * WCH, WCH), :] = \
                    stages[(c - 1) % 2][...].astype(jnp.bfloat16)
        cps[ncast - 1].wait()
        wv_ref[pl.ds((ncast - 1) * WCH, WCH), :] = \
            stages[(ncast - 1) % 2][...].astype(jnp.bfloat16)

    xf = xbf_ref[...]
    xb = xf.astype(jnp.bfloat16)
    tl = jnp.sum(xf * tw_ref[...],
                 axis=1, keepdims=True) - jnp.log(tf_ref[...])
    lab = lab_ref[...]

    for n in range(S // 1024):
        w = wv_ref[pl.ds(1024 * n, 1024), :]
        acc = lax.dot_general(xb, w, (((1,), (1,)), ((), ())),
                              preferred_element_type=jnp.float32)
        acc = acc - jnp.log(frq_ref[:, pl.ds(1024 * n, 1024)])
        acc = jnp.where(lab == ids_ref[:, pl.ds(1024 * n, 1024)],
                        jnp.float32(-1e37), acc)
        if n == 0:
            col = lax.broadcasted_iota(jnp.int32, acc.shape, 1)
            acc = jnp.where(col == 0, tl, acc)
        out_ref[:, pl.ds(1024 * n, 1024)] = acc

    wt = wv_ref[pl.ds(S, 8), :]
    acct = lax.dot_general(xb, wt, (((1,), (1,)), ((), ())),
                           preferred_element_type=jnp.float32)
    acct = acct - jnp.log(frq_ref[:, pl.ds(S, 8)])
    acct = jnp.where(lab == ids_ref[:, pl.ds(S, 8)], jnp.float32(-1e37), acct)
    out_ref[:, pl.ds(S, 1)] = acct[:, 0:1]


def _main(xbf, big, labels_col, ids_row, frq_row, tf_col):
    return pl.pallas_call(
        _main_body,
        grid=(B // BM,),
        in_specs=[
            pl.BlockSpec((BM, D), lambda i: (i, 0)),
            pl.BlockSpec(memory_space=pl.ANY),
            pl.BlockSpec((BM, D), lambda i: (i + SPAD // BM, 0)),
            pl.BlockSpec((BM, 1), lambda i: (i, 0)),
            pl.BlockSpec((1, SPAD), lambda i: (0, 0)),
            pl.BlockSpec((1, SPAD), lambda i: (0, 0)),
            pl.BlockSpec((BM, 1), lambda i: (i, 0)),
        ],
        out_specs=pl.BlockSpec((BM, S + 1), lambda i: (i, 0)),
        out_shape=jax.ShapeDtypeStruct((B, S + 1), jnp.float32),
        scratch_shapes=[
            pltpu.VMEM((SPAD, D), jnp.bfloat16),
            pltpu.VMEM((WCH, D), jnp.float32),
            pltpu.VMEM((WCH, D), jnp.float32),
            pltpu.SemaphoreType.DMA,
            pltpu.SemaphoreType.DMA,
        ],
        compiler_params=pltpu.CompilerParams(
            dimension_semantics=("arbitrary",),
        ),
    )(xbf, big, big, labels_col, ids_row, frq_row, tf_col)


def kernel(inputs, labels, weight, sample_ids, true_freq, sample_freq):
    labels_i = labels.astype(jnp.int32)
    ids_all = jnp.concatenate([
        jnp.zeros((1,), jnp.int32),
        sample_ids.astype(jnp.int32),
        jnp.zeros((SPAD - S - 1,), jnp.int32),
        labels_i,
    ])
    big = _sc_gather()(weight, ids_all)

    frq_row = jnp.concatenate([
        jnp.ones((1,), jnp.float32),
        sample_freq,
        jnp.ones((SPAD - S - 1,), jnp.float32),
    ]).reshape(1, SPAD)

    logits = _main(inputs, big, labels_i.reshape(B, 1),
                   ids_all[:SPAD].reshape(1, SPAD), frq_row,
                   true_freq.reshape(B, 1))
    return logits, jnp.zeros((B,), labels.dtype)


# speedup vs baseline: 1.3734x; 1.3734x over previous
import functools

import jax
import jax.numpy as jnp
from jax import lax
from jax.experimental import pallas as pl
from jax.experimental.pallas import tpu as pltpu
from jax.experimental.pallas import tpu_sc as plsc

S = 8192
D = 1024
B = 4096
SPAD = 8448
NROWS = SPAD + B

NC = 2
NS = 16
NW = NC * NS
RPW = NROWS // NW
SCH = (40, 40, 40, 40, 40, 40, 40, 40, 40, 32)

BM = 256
WCH = 1056


def _sc_gather_body(table, ids, out, idx_v, rows0, rows1, rows2,
                    g0, g1, g2, w0, w1, w2):
    wid = lax.axis_index("s") * NC + lax.axis_index("c")
    base = wid * RPW
    pltpu.sync_copy(ids.at[pl.ds(base, RPW)], idx_v)
    bufs = (rows0, rows1, rows2)
    gsems = (g0, g1, g2)
    wsems = (w0, w1, w2)
    n = len(SCH)
    offs = [sum(SCH[:c]) for c in range(n)]
    g_cps = [None] * n
    wb_cps = [None] * n
    for c in range(n):
        if c >= 3:
            wb_cps[c - 3].wait()
        b = bufs[c % 3].at[pl.ds(0, SCH[c])]
        g_cps[c] = pltpu.make_async_copy(
            table.at[idx_v.at[pl.ds(offs[c], SCH[c])]], b, gsems[c % 3])
        g_cps[c].start()
        if c >= 1:
            g_cps[c - 1].wait()
            pb = bufs[(c - 1) % 3].at[pl.ds(0, SCH[c - 1])]
            wb_cps[c - 1] = pltpu.make_async_copy(
                pb, out.at[pl.ds(base + offs[c - 1], SCH[c - 1])],
                wsems[(c - 1) % 3])
            wb_cps[c - 1].start()
    g_cps[n - 1].wait()
    wb_cps[n - 1] = pltpu.make_async_copy(
        bufs[(n - 1) % 3].at[pl.ds(0, SCH[n - 1])],
        out.at[pl.ds(base + offs[n - 1], SCH[n - 1])], wsems[(n - 1) % 3])
    wb_cps[n - 1].start()
    for c in (n - 3, n - 2, n - 1):
        wb_cps[c].wait()


@functools.cache
def _sc_gather():
    return pl.kernel(
        _sc_gather_body,
        out_type=jax.ShapeDtypeStruct((NROWS, D), jnp.float32),
        mesh=plsc.VectorSubcoreMesh(core_axis_name="c", subcore_axis_name="s"),
        scratch_types=[
            pltpu.VMEM((RPW,), jnp.int32),
            pltpu.VMEM((40, D), jnp.float32),
            pltpu.VMEM((40, D), jnp.float32),
            pltpu.VMEM((40, D), jnp.float32),
            pltpu.SemaphoreType.DMA,
            pltpu.SemaphoreType.DMA,
            pltpu.SemaphoreType.DMA,
            pltpu.SemaphoreType.DMA,
            pltpu.SemaphoreType.DMA,
            pltpu.SemaphoreType.DMA,
        ],
    )


def _main_body(xbf_ref, whbm_ref, tw_ref, lab_ref, ids_ref, frq_ref, tf_ref,
               out_ref, wv_ref, stage_ref, stage2_ref, sem, sem2):
    i = pl.program_id(0)

    @pl.when(i == 0)
    def _():
        stages = (stage_ref, stage2_ref)
        sems = (sem, sem2)
        ncast = SPAD // WCH
        cps = [None] * ncast
        for c in range(ncast):
            cps[c] = pltpu.make_async_copy(
                whbm_ref.at[pl.ds(c * WCH, WCH)], stages[c % 2], sems[c % 2])
            cps[c].start()
            if c >= 1:
                cps[c - 1].wait()
                wv_ref[pl.ds((c - 1) * WCH, WCH), :] = \
                    stages[(c - 1) % 2][...].astype(jnp.bfloat16)
        cps[ncast - 1].wait()
        wv_ref[pl.ds((ncast - 1) * WCH, WCH), :] = \
            stages[(ncast - 1) % 2][...].astype(jnp.bfloat16)

    xf = xbf_ref[...]
    xb = xf.astype(jnp.bfloat16)
    tl = jnp.sum(xf * tw_ref[...],
                 axis=1, keepdims=True) - jnp.log(tf_ref[...])
    lab = lab_ref[...]

    for n in range(S // 1024):
        w = wv_ref[pl.ds(1024 * n, 1024), :]
        acc = lax.dot_general(xb, w, (((1,), (1,)), ((), ())),
                              preferred_element_type=jnp.float32)
        acc = acc - jnp.log(frq_ref[:, pl.ds(1024 * n, 1024)])
        acc = jnp.where(lab == ids_ref[:, pl.ds(1024 * n, 1024)],
                        jnp.float32(-1e37), acc)
        if n == 0:
            col = lax.broadcasted_iota(jnp.int32, acc.shape, 1)
            acc = jnp.where(col == 0, tl, acc)
        out_ref[:, pl.ds(1024 * n, 1024)] = acc

    wt = wv_ref[pl.ds(S, 8), :]
    acct = lax.dot_general(xb, wt, (((1,), (1,)), ((), ())),
                           preferred_element_type=jnp.float32)
    acct = acct - jnp.log(frq_ref[:, pl.ds(S, 8)])
    acct = jnp.where(lab == ids_ref[:, pl.ds(S, 8)], jnp.float32(-1e37), acct)
    out_ref[:, pl.ds(S, 1)] = acct[:, 0:1]


def _main(xbf, big, labels_col, ids_row, frq_row, tf_col):
    return pl.pallas_call(
        _main_body,
        grid=(B // BM,),
        in_specs=[
            pl.BlockSpec((BM, D), lambda i: (i, 0)),
            pl.BlockSpec(memory_space=pl.ANY),
            pl.BlockSpec((BM, D), lambda i: (i + SPAD // BM, 0)),
            pl.BlockSpec((BM, 1), lambda i: (i, 0)),
            pl.BlockSpec((1, SPAD), lambda i: (0, 0)),
            pl.BlockSpec((1, SPAD), lambda i: (0, 0)),
            pl.BlockSpec((BM, 1), lambda i: (i, 0)),
        ],
        out_specs=pl.BlockSpec((BM, S + 1), lambda i: (i, 0)),
        out_shape=jax.ShapeDtypeStruct((B, S + 1), jnp.float32),
        scratch_shapes=[
            pltpu.VMEM((SPAD, D), jnp.bfloat16),
            pltpu.VMEM((WCH, D), jnp.float32),
            pltpu.VMEM((WCH, D), jnp.float32),
            pltpu.SemaphoreType.DMA,
            pltpu.SemaphoreType.DMA,
        ],
        compiler_params=pltpu.CompilerParams(
            dimension_semantics=("arbitrary",),
        ),
    )(xbf, big, big, labels_col, ids_row, frq_row, tf_col)


def kernel(inputs, labels, weight, sample_ids, true_freq, sample_freq):
    labels_i = labels.astype(jnp.int32)
    ids_all = jnp.concatenate([
        jnp.zeros((1,), jnp.int32),
        sample_ids.astype(jnp.int32),
        jnp.zeros((SPAD - S - 1,), jnp.int32),
        labels_i,
    ])
    big = _sc_gather()(weight, ids_all)

    frq_row = jnp.concatenate([
        jnp.ones((1,), jnp.float32),
        sample_freq,
        jnp.ones((SPAD - S - 1,), jnp.float32),
    ]).reshape(1, SPAD)

    logits = _main(inputs, big, labels_i.reshape(B, 1),
                   ids_all[:SPAD].reshape(1, SPAD), frq_row,
                   true_freq.reshape(B, 1))
    return logits, jnp.zeros((B,), labels.dtype)
